# baseline (speedup 1.0000x reference)
import functools

import jax
import jax.numpy as jnp
from jax import lax
from jax.experimental import pallas as pl
from jax.experimental.pallas import tpu as pltpu
from jax.experimental.pallas import tpu_sc as plsc

_CAPACITY = 0.5
_ROWS_SC = 2048
_NTILES = 32
_BPT = _ROWS_SC // 8 // _NTILES


def _score_body(hs_ref, w_ref, b_ref, out_ref):
    wrep = w_ref[...]
    x = hs_ref[0].astype(jnp.bfloat16)
    acc = lax.dot_general(
        wrep, x, (((1,), (1,)), ((), ())),
        preferred_element_type=jnp.float32)
    out_ref[0, 0, :] = jax.nn.sigmoid(acc[0, :] + b_ref[0])


def _round_bf16(v):
    u = lax.bitcast_convert_type(v, jnp.int32)
    r = (u + jnp.int32(0x7FFF) + ((u >> 16) & 1)) & jnp.int32(-65536)
    return lax.bitcast_convert_type(r, jnp.float32)


def _sc_body(row0, hs_ref, w_ref, out_ref, xb, wb, ob):
    wid = lax.axis_index("s") * 2 + lax.axis_index("c")
    pltpu.sync_copy(w_ref, wb)

    def band_loop(bi, _):
        band = wid * _BPT + bi
        pltpu.sync_copy(hs_ref.at[pl.ds(row0 + band * 8, 8)], xb)
        for s_row in range(8):
            def c_loop(c, acc):
                for jj in range(4):
                    xv = xb[s_row, pl.ds((c * 4 + jj) * 16, 16)]
                    wv = wb[pl.ds((c * 4 + jj) * 16, 16)]
                    acc = acc + _round_bf16(xv) * wv
                return acc
            acc = lax.fori_loop(0, 16, c_loop, jnp.zeros((16,), jnp.float32))
            ob[pl.ds((bi * 8 + s_row) * 16, 16)] = acc
        return 0

    lax.fori_loop(0, _BPT, band_loop, 0)
    pltpu.sync_copy(
        ob, out_ref.at[pl.ds(wid * _BPT * 8 * 16, _BPT * 8 * 16)])


def _finalize_body(k, main_ref, tail_ref, scp_ref, b_ref, w_out_ref, mask_ref):
    sc_logits = jnp.sum(scp_ref[...], axis=1)
    wsc = jax.nn.sigmoid(sc_logits + b_ref[0])
    last = jnp.concatenate(
        [tail_ref[...], wsc.reshape(1, -1)], axis=1)
    w = jnp.concatenate([main_ref[...], last], axis=0)
    keys = lax.bitcast_convert_type(w, jnp.int32)

    def body(j, t):
        bit = lax.shift_left(jnp.int32(1), jnp.int32(30) - j)
        cand = t | bit
        cnt = jnp.sum((keys >= cand).astype(jnp.int32),
                      axis=1, keepdims=True)
        return jnp.where(cnt >= k, cand, t)

    t = lax.fori_loop(0, 31, body,
                      jnp.zeros((w.shape[0], 1), jnp.int32))
    thr = lax.bitcast_convert_type(t, jnp.float32)
    w_out_ref[...] = w
    mask_ref[...] = (w >= thr).astype(jnp.int8)


def kernel(hidden_states, W, b):
    B, S, D = hidden_states.shape
    k = max(1, int(_CAPACITY * S))

    BS = 2048
    n_tc = B * S - _ROWS_SC
    n_blk = n_tc // BS
    hs3 = hidden_states.reshape((B * S) // BS, BS, D)
    hs2 = hidden_states.reshape(B * S, D)
    wrep = jnp.broadcast_to(W.astype(jnp.bfloat16), (8, D))
    w_rounded = W.astype(jnp.bfloat16).astype(jnp.float32).reshape(D)

    wtc3 = pl.pallas_call(
        _score_body,
        grid=(n_blk,),
        in_specs=[
            pl.BlockSpec((1, BS, D), lambda i: (i, 0, 0)),
            pl.BlockSpec((8, D), lambda i: (0, 0)),
            pl.BlockSpec((1,), lambda i: (0,)),
        ],
        out_specs=pl.BlockSpec((1, 1, BS), lambda i: (i, 0, 0)),
        out_shape=jax.ShapeDtypeStruct((n_blk, 1, BS), jnp.float32),
        compiler_params=pltpu.CompilerParams(
            dimension_semantics=("parallel",)),
    )(hs3, wrep, b)

    sc_score = functools.partial(
        pl.kernel,
        out_type=jax.ShapeDtypeStruct((_ROWS_SC * 16,), jnp.float32),
        mesh=plsc.VectorSubcoreMesh(core_axis_name="c", subcore_axis_name="s"),
        scratch_types=[
            pltpu.VMEM((8, D), jnp.float32),
            pltpu.VMEM((D,), jnp.float32),
            pltpu.VMEM((_BPT * 8 * 16,), jnp.float32),
        ],
    )(functools.partial(_sc_body, n_tc))
    sc_partials = sc_score(hs2, w_rounded).reshape(_ROWS_SC, 16)

    wtc_flat = wtc3.reshape(n_tc)
    n_main = (B - 1) * S
    wtc_main = wtc_flat[:n_main].reshape(B - 1, S)
    wtc_tail = wtc_flat[n_main:].reshape(1, S - _ROWS_SC)

    weights, mask_i8 = pl.pallas_call(
        functools.partial(_finalize_body, k),
        in_specs=[
            pl.BlockSpec((B - 1, S), lambda: (0, 0)),
            pl.BlockSpec((1, S - _ROWS_SC), lambda: (0, 0)),
            pl.BlockSpec((_ROWS_SC, 16), lambda: (0, 0)),
            pl.BlockSpec((1,), lambda: (0,)),
        ],
        out_specs=[
            pl.BlockSpec((B, S), lambda: (0, 0)),
            pl.BlockSpec((B, S), lambda: (0, 0)),
        ],
        out_shape=[
            jax.ShapeDtypeStruct((B, S), jnp.float32),
            jax.ShapeDtypeStruct((B, S), jnp.int8),
        ],
    )(wtc_main, wtc_tail, sc_partials, b)

    return weights, mask_i8.astype(bool)


# speedup vs baseline: 1.0003x; 1.0003x over previous
import functools

import jax
import jax.numpy as jnp
from jax import lax
from jax.experimental import pallas as pl
from jax.experimental.pallas import tpu as pltpu
from jax.experimental.pallas import tpu_sc as plsc

_CAPACITY = 0.5
_ROWS_SC = 2048
_NTILES = 32
_BPT = _ROWS_SC // 8 // _NTILES


def _score_body(hs_ref, w_ref, b_ref, out_ref):
    wrep = w_ref[...]
    x = hs_ref[0].astype(jnp.bfloat16)
    acc = lax.dot_general(
        wrep, x, (((1,), (1,)), ((), ())),
        preferred_element_type=jnp.float32)
    out_ref[0, 0, :] = jax.nn.sigmoid(acc[0, :] + b_ref[0])


def _round_bf16(v):
    u = lax.bitcast_convert_type(v, jnp.int32)
    r = (u + jnp.int32(0x7FFF) + ((u >> 16) & 1)) & jnp.int32(-65536)
    return lax.bitcast_convert_type(r, jnp.float32)


def _sc_body(row0, hs_ref, w_ref, out_ref, xb, wb, ob):
    wid = lax.axis_index("s") * 2 + lax.axis_index("c")
    pltpu.sync_copy(w_ref, wb)

    def band_loop(bi, _):
        band = wid * _BPT + bi
        pltpu.sync_copy(hs_ref.at[pl.ds(row0 + band * 8, 8)], xb)
        for s_row in range(8):
            def c_loop(c, acc):
                for jj in range(4):
                    xv = xb[s_row, pl.ds((c * 4 + jj) * 16, 16)]
                    wv = wb[pl.ds((c * 4 + jj) * 16, 16)]
                    acc = acc + _round_bf16(xv) * wv
                return acc
            acc = lax.fori_loop(0, 16, c_loop, jnp.zeros((16,), jnp.float32))
            ob[pl.ds((bi * 8 + s_row) * 16, 16)] = acc
        return 0

    lax.fori_loop(0, _BPT, band_loop, 0)
    pltpu.sync_copy(
        ob, out_ref.at[pl.ds(wid * _BPT * 8 * 16, _BPT * 8 * 16)])


def _finalize_body(k, main_ref, tail_ref, scp_ref, b_ref, w_out_ref, mask_ref):
    sc_logits = jnp.sum(scp_ref[...], axis=1)
    wsc = jax.nn.sigmoid(sc_logits + b_ref[0])
    last = jnp.concatenate(
        [tail_ref[...], wsc.reshape(1, -1)], axis=1)
    w = jnp.concatenate([main_ref[...], last], axis=0)
    keys = lax.bitcast_convert_type(w, jnp.int32)

    def body(j, t):
        bit = lax.shift_left(jnp.int32(1), jnp.int32(30) - j)
        cand = t | bit
        cnt = jnp.sum((keys >= cand).astype(jnp.int32),
                      axis=1, keepdims=True)
        return jnp.where(cnt >= k, cand, t)

    t = lax.fori_loop(0, 31, body,
                      jnp.zeros((w.shape[0], 1), jnp.int32))
    thr = lax.bitcast_convert_type(t, jnp.float32)
    w_out_ref[...] = w
    mask_ref[...] = (w >= thr).astype(jnp.int8)


def kernel(hidden_states, W, b):
    B, S, D = hidden_states.shape
    k = max(1, int(_CAPACITY * S))

    BS = 2048
    n_tc = B * S - _ROWS_SC
    n_blk = n_tc // BS
    hs3 = hidden_states.reshape((B * S) // BS, BS, D)
    hs2 = hidden_states.reshape(B * S, D)
    wrep = jnp.broadcast_to(W.astype(jnp.bfloat16), (8, D))
    w_rounded = W.astype(jnp.bfloat16).astype(jnp.float32).reshape(D)

    sc_score = functools.partial(
        pl.kernel,
        out_type=jax.ShapeDtypeStruct((_ROWS_SC * 16,), jnp.float32),
        mesh=plsc.VectorSubcoreMesh(core_axis_name="c", subcore_axis_name="s"),
        scratch_types=[
            pltpu.VMEM((8, D), jnp.float32),
            pltpu.VMEM((D,), jnp.float32),
            pltpu.VMEM((_BPT * 8 * 16,), jnp.float32),
        ],
    )(functools.partial(_sc_body, n_tc))
    sc_partials = sc_score(hs2, w_rounded).reshape(_ROWS_SC, 16)

    wtc3 = pl.pallas_call(
        _score_body,
        grid=(n_blk,),
        in_specs=[
            pl.BlockSpec((1, BS, D), lambda i: (i, 0, 0)),
            pl.BlockSpec((8, D), lambda i: (0, 0)),
            pl.BlockSpec((1,), lambda i: (0,)),
        ],
        out_specs=pl.BlockSpec((1, 1, BS), lambda i: (i, 0, 0)),
        out_shape=jax.ShapeDtypeStruct((n_blk, 1, BS), jnp.float32),
        compiler_params=pltpu.CompilerParams(
            dimension_semantics=("parallel",)),
    )(hs3, wrep, b)

    wtc_flat = wtc3.reshape(n_tc)
    n_main = (B - 1) * S
    wtc_main = wtc_flat[:n_main].reshape(B - 1, S)
    wtc_tail = wtc_flat[n_main:].reshape(1, S - _ROWS_SC)

    weights, mask_i8 = pl.pallas_call(
        functools.partial(_finalize_body, k),
        in_specs=[
            pl.BlockSpec((B - 1, S), lambda: (0, 0)),
            pl.BlockSpec((1, S - _ROWS_SC), lambda: (0, 0)),
            pl.BlockSpec((_ROWS_SC, 16), lambda: (0, 0)),
            pl.BlockSpec((1,), lambda: (0,)),
        ],
        out_specs=[
            pl.BlockSpec((B, S), lambda: (0, 0)),
            pl.BlockSpec((B, S), lambda: (0, 0)),
        ],
        out_shape=[
            jax.ShapeDtypeStruct((B, S), jnp.float32),
            jax.ShapeDtypeStruct((B, S), jnp.int8),
        ],
    )(wtc_main, wtc_tail, sc_partials, b)

    return weights, mask_i8.astype(bool)


# speedup vs baseline: 1.4319x; 1.4314x over previous
import functools

import jax
import jax.numpy as jnp
from jax.experimental import pallas as pl
from jax.experimental.pallas import tpu as pltpu

_CAPACITY = 0.5


def _fused_body(k, n_blk, blk_per_row, hs_ref, w_ref, b_ref,
                out_ref, mask_ref, wacc_ref):
    i = pl.program_id(0)
    BS = out_ref.shape[2]

    wrep = w_ref[...]
    x = hs_ref[0].astype(jnp.bfloat16)
    acc = jax.lax.dot_general(
        wrep, x, (((1,), (1,)), ((), ())),
        preferred_element_type=jnp.float32)
    wts = jax.nn.sigmoid(acc[0:1, :] + b_ref[0])
    out_ref[0, 0, :] = wts[0]

    row = i // blk_per_row
    off = (i % blk_per_row) * BS
    wacc_ref[pl.ds(row, 1), pl.ds(off, BS)] = wts

    @pl.when(i == n_blk - 1)
    def _select():
        w = wacc_ref[...]
        keys = jax.lax.bitcast_convert_type(w, jnp.int32)
        nrow = w.shape[0]

        def body(j, t):
            bit = jax.lax.shift_left(jnp.int32(1), jnp.int32(30) - j)
            cand = t | bit
            cnt = jnp.sum((keys >= cand).astype(jnp.int32),
                          axis=1, keepdims=True)
            return jnp.where(cnt >= k, cand, t)

        t = jax.lax.fori_loop(0, 31, body, jnp.zeros((nrow, 1), jnp.int32))
        thr = jax.lax.bitcast_convert_type(t, jnp.float32)
        mask_ref[...] = (w >= thr).astype(jnp.int8)


def kernel(hidden_states, W, b):
    B, S, D = hidden_states.shape
    k = max(1, int(_CAPACITY * S))

    BS = 2048
    n_blk = (B * S) // BS
    blk_per_row = S // BS
    hs3 = hidden_states.reshape(n_blk, BS, D)
    wrep = jnp.broadcast_to(W.astype(jnp.bfloat16), (8, D))

    weights3, mask_i8 = pl.pallas_call(
        functools.partial(_fused_body, k, n_blk, blk_per_row),
        grid=(n_blk,),
        in_specs=[
            pl.BlockSpec((1, BS, D), lambda i: (i, 0, 0)),
            pl.BlockSpec((8, D), lambda i: (0, 0)),
            pl.BlockSpec((1,), lambda i: (0,)),
        ],
        out_specs=[
            pl.BlockSpec((1, 1, BS), lambda i: (i, 0, 0)),
            pl.BlockSpec((B, S), lambda i: (0, 0)),
        ],
        out_shape=[
            jax.ShapeDtypeStruct((n_blk, 1, BS), jnp.float32),
            jax.ShapeDtypeStruct((B, S), jnp.int8),
        ],
        scratch_shapes=[pltpu.VMEM((B, S), jnp.float32)],
        compiler_params=pltpu.CompilerParams(
            dimension_semantics=("arbitrary",)),
    )(hs3, wrep, b)

    return weights3.reshape(B, S), mask_i8.astype(bool)


# speedup vs baseline: 1.4924x; 1.0423x over previous
import functools

import jax
import jax.numpy as jnp
from jax.experimental import pallas as pl
from jax.experimental.pallas import tpu as pltpu

_CAPACITY = 0.5


def _fused_body(k, n_blk, blk_per_row, hs_ref, w_ref, b_ref,
                w_out_ref, mask_ref):
    i = pl.program_id(0)
    BS = hs_ref.shape[1]

    wrep = w_ref[...]
    x = hs_ref[0].astype(jnp.bfloat16)
    acc = jax.lax.dot_general(
        wrep, x, (((1,), (1,)), ((), ())),
        preferred_element_type=jnp.float32)
    wts = jax.nn.sigmoid(acc[0:1, :] + b_ref[0])

    row = i // blk_per_row
    off = (i % blk_per_row) * BS
    w_out_ref[pl.ds(row, 1), pl.ds(off, BS)] = wts

    @pl.when(i == n_blk - 1)
    def _select():
        w = w_out_ref[...]
        keys = jax.lax.bitcast_convert_type(w, jnp.int32)
        nrow = w.shape[0]

        def body(j, t):
            bit = jax.lax.shift_left(jnp.int32(1), jnp.int32(30) - j)
            cand = t | bit
            cnt = jnp.sum((keys >= cand).astype(jnp.int32),
                          axis=1, keepdims=True)
            return jnp.where(cnt >= k, cand, t)

        t = jax.lax.fori_loop(0, 31, body, jnp.zeros((nrow, 1), jnp.int32))
        thr = jax.lax.bitcast_convert_type(t, jnp.float32)
        mask_ref[...] = (w >= thr).astype(jnp.int8)


def kernel(hidden_states, W, b):
    B, S, D = hidden_states.shape
    k = max(1, int(_CAPACITY * S))

    BS = 2048
    n_blk = (B * S) // BS
    blk_per_row = S // BS
    hs3 = hidden_states.reshape(n_blk, BS, D)
    wrep = jnp.broadcast_to(W.astype(jnp.bfloat16), (8, D))

    weights, mask_i8 = pl.pallas_call(
        functools.partial(_fused_body, k, n_blk, blk_per_row),
        grid=(n_blk,),
        in_specs=[
            pl.BlockSpec((1, BS, D), lambda i: (i, 0, 0)),
            pl.BlockSpec((8, D), lambda i: (0, 0)),
            pl.BlockSpec((1,), lambda i: (0,)),
        ],
        out_specs=[
            pl.BlockSpec((B, S), lambda i: (0, 0)),
            pl.BlockSpec((B, S), lambda i: (0, 0)),
        ],
        out_shape=[
            jax.ShapeDtypeStruct((B, S), jnp.float32),
            jax.ShapeDtypeStruct((B, S), jnp.int8),
        ],
        compiler_params=pltpu.CompilerParams(
            dimension_semantics=("arbitrary",)),
    )(hs3, wrep, b)

    return weights, mask_i8.astype(bool)
